# baseline (speedup 1.0000x reference)
import jax
import jax.numpy as jnp
from jax.experimental import pallas as pl
from jax.experimental.pallas import tpu as pltpu


def _router_body(x_ref, m_ref, w_ref, probs_ref, logits_ref):
    x = x_ref[...]
    w = w_ref[...]
    logits = jnp.dot(x, w, preferred_element_type=jnp.float32)
    mx = jnp.max(logits, axis=-1, keepdims=True)
    e = jnp.exp(logits - mx)
    s = jnp.sum(e, axis=-1, keepdims=True)
    probs_ref[...] = (e / s) * m_ref[...]
    logits_ref[...] = logits


def kernel(inputs, padding_mask, w, num_experts):
    T, D = inputs.shape
    E = w.shape[1]
    BT = 512
    probs, logits = pl.pallas_call(
        _router_body,
        grid=(T // BT,),
        compiler_params=pltpu.CompilerParams(
            dimension_semantics=("arbitrary",),
        ),
        in_specs=[
            pl.BlockSpec((BT, D), lambda i: (i, 0)),
            pl.BlockSpec((BT, 1), lambda i: (i, 0)),
            pl.BlockSpec((D, E), lambda i: (0, 0)),
        ],
        out_specs=[
            pl.BlockSpec((BT, E), lambda i: (i, 0)),
            pl.BlockSpec((BT, E), lambda i: (i, 0)),
        ],
        out_shape=[
            jax.ShapeDtypeStruct((T, E), jnp.float32),
            jax.ShapeDtypeStruct((T, E), jnp.float32),
        ],
    )(inputs, padding_mask, w)
    return (probs, logits)


# speedup vs baseline: 1.4091x; 1.4091x over previous
import jax
import jax.numpy as jnp
from jax.experimental import pallas as pl
from jax.experimental.pallas import tpu as pltpu


def _router_body(x_ref, m_ref, w_ref, probs_ref, logits_ref):
    x = x_ref[...]
    w = w_ref[...]
    logits = jnp.dot(x, w, preferred_element_type=jnp.float32)
    mx = jnp.max(logits, axis=-1, keepdims=True)
    e = jnp.exp(logits - mx)
    s = jnp.sum(e, axis=-1, keepdims=True)
    probs_ref[...] = (e / s) * m_ref[...]
    logits_ref[...] = logits


def kernel(inputs, padding_mask, w, num_experts):
    T, D = inputs.shape
    E = w.shape[1]
    BT = 4096
    probs, logits = pl.pallas_call(
        _router_body,
        grid=(T // BT,),
        compiler_params=pltpu.CompilerParams(
            dimension_semantics=("arbitrary",),
        ),
        in_specs=[
            pl.BlockSpec((BT, D), lambda i: (i, 0)),
            pl.BlockSpec((BT, 1), lambda i: (i, 0)),
            pl.BlockSpec((D, E), lambda i: (0, 0)),
        ],
        out_specs=[
            pl.BlockSpec((BT, E), lambda i: (i, 0)),
            pl.BlockSpec((BT, E), lambda i: (i, 0)),
        ],
        out_shape=[
            jax.ShapeDtypeStruct((T, E), jnp.float32),
            jax.ShapeDtypeStruct((T, E), jnp.float32),
        ],
    )(inputs, padding_mask, w)
    return (probs, logits)
